# Initial kernel scaffold; baseline (speedup 1.0000x reference)
import jax
import jax.numpy as jnp
from jax.experimental import pallas as pl


def kernel(z, cluster_logits, temperature, centroids):
    raise NotImplementedError("write your pallas kernel here")


# speedup vs baseline: 1.1240x; 1.1240x over previous
import functools

import jax
import jax.numpy as jnp
import numpy as np
from jax.experimental import pallas as pl
from jax.experimental.pallas import tpu as pltpu

_B, _T, _K, _D = 16, 1024, 1024, 64
_TT = 512

_K0 = np.uint32(0)
_K1 = np.uint32(42)
_K2 = np.uint32(0 ^ 42 ^ 0x1BD11BDA)
_ROT = ((13, 15, 26, 6), (17, 29, 16, 24))
_TINY = np.float32(np.finfo(np.float32).tiny)


def _threefry_bits(idx_u32):
    ks = (_K0, _K1, _K2)
    x0 = jnp.full_like(idx_u32, ks[0])
    x1 = idx_u32 + ks[1]
    for i in range(5):
        for r in _ROT[i % 2]:
            x0 = x0 + x1
            x1 = (x1 << np.uint32(r)) | (x1 >> np.uint32(32 - r))
            x1 = x1 ^ x0
        x0 = x0 + ks[(i + 1) % 3]
        x1 = x1 + ks[(i + 2) % 3] + np.uint32(i + 1)
    return x0 ^ x1


def _gumbel_from_bits(bits):
    fb = (bits >> np.uint32(9)) | np.uint32(0x3F800000)
    u01 = jax.lax.bitcast_convert_type(fb, jnp.float32) - jnp.float32(1.0)
    u = jnp.maximum(u01 * (jnp.float32(1.0) - _TINY) + _TINY, _TINY)
    return -jnp.log(-jnp.log(u))


def _loss_kernel(logits_ref, z_ref, cent_ref, out_ref):
    b = pl.program_id(0)
    t = pl.program_id(1)

    row = jax.lax.broadcasted_iota(jnp.int32, (_TT, _K), 0)
    col = jax.lax.broadcasted_iota(jnp.int32, (_TT, _K), 1)
    base = (b * _T + t * _TT) * _K
    lin = (base + row * _K + col).astype(jnp.uint32)

    g = _gumbel_from_bits(_threefry_bits(lin))
    s = logits_ref[0] + g

    m = jnp.max(s, axis=1, keepdims=True)
    idx = jnp.min(jnp.where(s == m, col, _K), axis=1)

    one_hot = (col == idx[:, None]).astype(jnp.float32)
    q = jnp.dot(one_hot, cent_ref[...], preferred_element_type=jnp.float32)
    diff = z_ref[0] - q
    part = jnp.sum(diff * diff)

    @pl.when((b == 0) & (t == 0))
    def _():
        out_ref[0, 0] = jnp.float32(0.0)

    acc = out_ref[0, 0] + part
    out_ref[0, 0] = acc

    @pl.when((b == _B - 1) & (t == (_T // _TT) - 1))
    def _():
        out_ref[0, 0] = acc / jnp.float32(_B * _T * _D)


@functools.partial(jax.jit, static_argnames=())
def _run(z, cluster_logits, centroids):
    out = pl.pallas_call(
        _loss_kernel,
        grid=(_B, _T // _TT),
        in_specs=[
            pl.BlockSpec((1, _TT, _K), lambda b, t: (b, t, 0)),
            pl.BlockSpec((1, _TT, _D), lambda b, t: (b, t, 0)),
            pl.BlockSpec((_K, _D), lambda b, t: (0, 0)),
        ],
        out_specs=pl.BlockSpec(memory_space=pltpu.SMEM),
        out_shape=jax.ShapeDtypeStruct((1, 1), jnp.float32),
    )(cluster_logits, z, centroids)
    return out[0, 0]


def kernel(z, cluster_logits, temperature, centroids):
    del temperature
    return _run(z, cluster_logits, centroids)


# speedup vs baseline: 1.1593x; 1.0314x over previous
import functools

import jax
import jax.numpy as jnp
import numpy as np
from jax.experimental import pallas as pl
from jax.experimental.pallas import tpu as pltpu

_B, _T, _K, _D = 16, 1024, 1024, 64
_TT = 512

_K0 = np.uint32(0)
_K1 = np.uint32(42)
_K2 = np.uint32(0 ^ 42 ^ 0x1BD11BDA)
_ROT = ((13, 15, 26, 6), (17, 29, 16, 24))
_TINY = np.float32(np.finfo(np.float32).tiny)


def _threefry_bits(x1_keyed):
    ks = (_K0, _K1, _K2)
    x1 = x1_keyed
    x0 = x1
    x1 = ((x1 << np.uint32(13)) | (x1 >> np.uint32(19))) ^ x0
    for r in _ROT[0][1:]:
        x0 = x0 + x1
        x1 = (x1 << np.uint32(r)) | (x1 >> np.uint32(32 - r))
        x1 = x1 ^ x0
    x0 = x0 + ks[1]
    x1 = x1 + np.uint32(ks[2] + np.uint32(1))
    for i in range(1, 5):
        for r in _ROT[i % 2]:
            x0 = x0 + x1
            x1 = (x1 << np.uint32(r)) | (x1 >> np.uint32(32 - r))
            x1 = x1 ^ x0
        x0 = x0 + ks[(i + 1) % 3]
        x1 = x1 + np.uint32(ks[(i + 2) % 3] + np.uint32(i + 1))
    return x0 ^ x1


def _gumbel_from_bits(bits):
    fb = (bits >> np.uint32(9)) | np.uint32(0x3F800000)
    u01 = jax.lax.bitcast_convert_type(fb, jnp.float32) - jnp.float32(1.0)
    u = u01 + _TINY
    return -jnp.log(-jnp.log(u))


def _loss_kernel(logits_ref, z_ref, cent_ref, out_ref):
    b = pl.program_id(0)
    t = pl.program_id(1)

    row = jax.lax.broadcasted_iota(jnp.uint32, (_TT, _K), 0)
    col = jax.lax.broadcasted_iota(jnp.int32, (_TT, _K), 1)
    base = ((b * _T + t * _TT) * _K).astype(jnp.uint32) + _K1
    x1 = (row << np.uint32(10)) + col.astype(jnp.uint32) + base

    g = _gumbel_from_bits(_threefry_bits(x1))
    s = logits_ref[0] + g

    m = jnp.max(s, axis=1, keepdims=True)
    idx = jnp.min(jnp.where(s == m, col, _K), axis=1)

    one_hot = (col == idx[:, None]).astype(jnp.float32)
    q = jnp.dot(one_hot, cent_ref[...], preferred_element_type=jnp.float32)
    diff = z_ref[0] - q
    part = jnp.sum(diff * diff)

    @pl.when((b == 0) & (t == 0))
    def _():
        out_ref[0, 0] = jnp.float32(0.0)

    acc = out_ref[0, 0] + part
    out_ref[0, 0] = acc

    @pl.when((b == _B - 1) & (t == (_T // _TT) - 1))
    def _():
        out_ref[0, 0] = acc / jnp.float32(_B * _T * _D)


@functools.partial(jax.jit, static_argnames=())
def _run(z, cluster_logits, centroids):
    out = pl.pallas_call(
        _loss_kernel,
        grid=(_B, _T // _TT),
        in_specs=[
            pl.BlockSpec((1, _TT, _K), lambda b, t: (b, t, 0)),
            pl.BlockSpec((1, _TT, _D), lambda b, t: (b, t, 0)),
            pl.BlockSpec((_K, _D), lambda b, t: (0, 0)),
        ],
        out_specs=pl.BlockSpec(memory_space=pltpu.SMEM),
        out_shape=jax.ShapeDtypeStruct((1, 1), jnp.float32),
    )(cluster_logits, z, centroids)
    return out[0, 0]


def kernel(z, cluster_logits, temperature, centroids):
    del temperature
    return _run(z, cluster_logits, centroids)


# speedup vs baseline: 1.1825x; 1.0200x over previous
import functools

import jax
import jax.numpy as jnp
import numpy as np
from jax.experimental import pallas as pl
from jax.experimental.pallas import tpu as pltpu

_B, _T, _K, _D = 16, 1024, 1024, 64
_TT = 1024

_K0 = np.uint32(0)
_K1 = np.uint32(42)
_K2 = np.uint32(0 ^ 42 ^ 0x1BD11BDA)
_ROT = ((13, 15, 26, 6), (17, 29, 16, 24))
_TINY = np.float32(np.finfo(np.float32).tiny)


def _threefry_bits(x1_keyed):
    ks = (_K0, _K1, _K2)
    x1 = x1_keyed
    x0 = x1
    x1 = ((x1 << np.uint32(13)) | (x1 >> np.uint32(19))) ^ x0
    for r in _ROT[0][1:]:
        x0 = x0 + x1
        x1 = (x1 << np.uint32(r)) | (x1 >> np.uint32(32 - r))
        x1 = x1 ^ x0
    x0 = x0 + ks[1]
    x1 = x1 + np.uint32(ks[2] + np.uint32(1))
    for i in range(1, 5):
        for r in _ROT[i % 2]:
            x0 = x0 + x1
            x1 = (x1 << np.uint32(r)) | (x1 >> np.uint32(32 - r))
            x1 = x1 ^ x0
        x0 = x0 + ks[(i + 1) % 3]
        x1 = x1 + np.uint32(ks[(i + 2) % 3] + np.uint32(i + 1))
    return x0 ^ x1


def _gumbel_from_bits(bits):
    fb = (bits >> np.uint32(9)) | np.uint32(0x3F800000)
    u01 = jax.lax.bitcast_convert_type(fb, jnp.float32) - jnp.float32(1.0)
    u = u01 + _TINY
    return -jnp.log(-jnp.log(u))


def _loss_kernel(logits_ref, z_ref, cent_ref, out_ref):
    b = pl.program_id(0)
    t = pl.program_id(1)

    row = jax.lax.broadcasted_iota(jnp.uint32, (_TT, _K), 0)
    col = jax.lax.broadcasted_iota(jnp.int32, (_TT, _K), 1)
    base = ((b * _T + t * _TT) * _K).astype(jnp.uint32) + _K1
    x1 = (row << np.uint32(10)) + col.astype(jnp.uint32) + base

    g = _gumbel_from_bits(_threefry_bits(x1))
    s = logits_ref[0] + g

    m = jnp.max(s, axis=1, keepdims=True)
    idx = jnp.min(jnp.where(s == m, col, _K), axis=1)

    one_hot = (col == idx[:, None]).astype(jnp.float32)
    q = jnp.dot(one_hot, cent_ref[...], preferred_element_type=jnp.float32)
    diff = z_ref[0] - q
    part = jnp.sum(diff * diff)

    @pl.when((b == 0) & (t == 0))
    def _():
        out_ref[0, 0] = jnp.float32(0.0)

    acc = out_ref[0, 0] + part
    out_ref[0, 0] = acc

    @pl.when((b == _B - 1) & (t == (_T // _TT) - 1))
    def _():
        out_ref[0, 0] = acc / jnp.float32(_B * _T * _D)


@functools.partial(jax.jit, static_argnames=())
def _run(z, cluster_logits, centroids):
    out = pl.pallas_call(
        _loss_kernel,
        grid=(_B, _T // _TT),
        in_specs=[
            pl.BlockSpec((1, _TT, _K), lambda b, t: (b, t, 0)),
            pl.BlockSpec((1, _TT, _D), lambda b, t: (b, t, 0)),
            pl.BlockSpec((_K, _D), lambda b, t: (0, 0)),
        ],
        out_specs=pl.BlockSpec(memory_space=pltpu.SMEM),
        out_shape=jax.ShapeDtypeStruct((1, 1), jnp.float32),
    )(cluster_logits, z, centroids)
    return out[0, 0]


def kernel(z, cluster_logits, temperature, centroids):
    del temperature
    return _run(z, cluster_logits, centroids)
